# Initial kernel scaffold; baseline (speedup 1.0000x reference)
import jax
import jax.numpy as jnp
from jax.experimental import pallas as pl


def kernel(adj_indices, adj_values, ui_indices, ui_values, user_ids, pos_item_ids, neg_item_ids, emb_user, emb_item, sem_table, W_proj, b_proj, W_gate, b_gate):
    raise NotImplementedError("write your pallas kernel here")


# speedup vs baseline: 6.4604x; 6.4604x over previous
import functools

import jax
import jax.numpy as jnp
from jax import lax
from jax.experimental import pallas as pl
from jax.experimental.pallas import tpu as pltpu
from jax.experimental.pallas import tpu_sc as plsc

NU = 50000
NI = 50000
N = NU + NI
D = 64
SEMD = 256
B = 4096
NC = 2
NS = 16
L = 16

CHUNK = 1024
KSUB = CHUNK // 128

_MESH = dict(core_axis_name="c", subcore_axis_name="s", num_cores=NC,
             num_subcores=NS)


def _splat(vec, idx):
    dn = lax.GatherDimensionNumbers(offset_dims=(), collapsed_slice_dims=(0,),
                                    start_index_map=(0,))
    return lax.gather(vec, idx[:, None], dn, slice_sizes=(1,),
                      mode=lax.GatherScatterMode.PROMISE_IN_BOUNDS)


def _spmm_planes(dst2, src2, val1, table_flat, n_rows, chunks_per_tile, zch,
                 plane_rows, normalize):
    n_zch = n_rows // zch
    wb_iters = (n_zch + NS - 1) // NS
    ept = chunks_per_tile * CHUNK

    scratch = [
        pltpu.VMEM_SHARED((n_rows, L), jnp.float32),
        pltpu.VMEM((CHUNK, L), jnp.float32),
        pltpu.VMEM((CHUNK,), jnp.float32),
        pltpu.VMEM((KSUB, 128), jnp.int32),
        pltpu.VMEM((KSUB, 128), jnp.int32),
        pltpu.SemaphoreType.DMA,
        pltpu.SemaphoreType.DMA,
    ]
    if normalize:
        scratch += [
            pltpu.VMEM_SHARED((n_rows,), jnp.float32),
            pltpu.VMEM((zch,), jnp.float32),
        ]

    def body(dst_h, src_h, val_h, tab_h, out_h, acc, rows, valb, gidx, dstb,
             gsem, ssem, *extra):
        c = lax.axis_index("c")
        s = lax.axis_index("s")
        iota = lax.iota(jnp.int32, L)
        cols = [iota * 0 + j for j in range(L)]
        if normalize:
            rsacc, rsbuf = extra

        for gl in range(2):
            g = c * 2 + gl
            off = g * plane_rows

            def zrow(i, _):
                rows[i, :] = jnp.zeros((L,), jnp.float32)
                return 0
            lax.fori_loop(0, zch, zrow, 0)
            if normalize and gl == 0:
                def zrs(i, _):
                    rsbuf[pl.ds(i * L, L)] = jnp.zeros((L,), jnp.float32)
                    return 0
                lax.fori_loop(0, zch // L, zrs, 0)
            for k in range(wb_iters):
                idx = s + NS * k

                @pl.when(idx < n_zch)
                def _():
                    pltpu.sync_copy(rows.at[pl.ds(0, zch)],
                                    acc.at[pl.ds(idx * zch, zch)])
                    if normalize and gl == 0:
                        pltpu.sync_copy(rsbuf, rsacc.at[pl.ds(idx * zch, zch)])
            plsc.subcore_barrier()

            def chunk_body(i, _):
                base = s * ept + i * CHUNK
                rb = s * (ept // 128) + i * KSUB
                pltpu.sync_copy(src_h.at[pl.ds(rb, KSUB)], gidx)
                pltpu.sync_copy(dst_h.at[pl.ds(rb, KSUB)], dstb)
                pltpu.sync_copy(val_h.at[pl.ds(base, CHUNK)], valb)
                for r in range(KSUB):
                    for c8 in range(8):
                        sl = pl.ds(c8 * L, L)
                        gidx[r, sl] = gidx[r, sl] + off
                gds = [
                    pltpu.async_copy(tab_h.at[gidx.at[j]],
                                     rows.at[pl.ds(j * 128, 128)], gsem)
                    for j in range(KSUB)
                ]
                for d in gds:
                    d.wait()
                def scale(b2, _):
                    rb16 = b2 * L
                    vv = valb[pl.ds(rb16, L)]
                    for j in range(L):
                        sp = _splat(vv, cols[j])
                        rows[rb16 + j, :] = rows[rb16 + j, :] * sp
                    return 0
                lax.fori_loop(0, CHUNK // L, scale, 0)
                sds = [
                    pltpu.async_copy(rows.at[pl.ds(j * 128, 128)],
                                     acc.at[dstb.at[j]], ssem, add=True)
                    for j in range(KSUB)
                ]
                if normalize and gl == 0:
                    sds += [
                        pltpu.async_copy(valb.at[pl.ds(j * 128, 128)],
                                         rsacc.at[dstb.at[j]], ssem, add=True)
                        for j in range(KSUB)
                    ]
                for d in sds:
                    d.wait()
                return 0
            lax.fori_loop(0, chunks_per_tile, chunk_body, 0)
            plsc.subcore_barrier()

            for k in range(wb_iters):
                idx = s + NS * k

                @pl.when(idx < n_zch)
                def _():
                    if normalize:
                        pltpu.sync_copy(acc.at[pl.ds(idx * zch, zch)],
                                        rows.at[pl.ds(0, zch)])
                        pltpu.sync_copy(rsacc.at[pl.ds(idx * zch, zch)], rsbuf)

                        def dv(b2, _):
                            rb16 = b2 * L
                            rsv = 1.0 / jnp.maximum(rsbuf[pl.ds(rb16, L)],
                                                    1.0)
                            for j in range(L):
                                sp = _splat(rsv, cols[j])
                                rows[rb16 + j, :] = rows[rb16 + j, :] * sp
                            return 0
                        lax.fori_loop(0, zch // L, dv, 0)
                        pltpu.sync_copy(rows.at[pl.ds(0, zch)],
                                        out_h.at[g, pl.ds(idx * zch, zch)])
                    else:
                        pltpu.sync_copy(acc.at[pl.ds(idx * zch, zch)],
                                        out_h.at[g, pl.ds(idx * zch, zch)])
            plsc.subcore_barrier()

    mesh = plsc.VectorSubcoreMesh(**_MESH)
    kern = pl.kernel(
        body,
        out_type=jax.ShapeDtypeStruct((4, n_rows, L), jnp.float32),
        mesh=mesh,
        scratch_types=scratch,
        compiler_params=pltpu.CompilerParams(use_tc_tiling_on_sc=False),
        name=f"spmm_sc_{n_rows}_{chunks_per_tile}",
    )
    return kern(dst2, src2, val1, table_flat)


def _batch_gather(all_flat, sem_tab, c_flat, uid, pid, nid):
    rows_per = B // (NC * NS)

    scratch = [
        pltpu.VMEM((17, 128), jnp.int32),
        pltpu.VMEM((4, rows_per, L), jnp.float32),
        pltpu.VMEM((4, rows_per, L), jnp.float32),
        pltpu.VMEM((4, rows_per, L), jnp.float32),
        pltpu.VMEM((4, rows_per, L), jnp.float32),
        pltpu.VMEM((rows_per, SEMD), jnp.float32),
        pltpu.SemaphoreType.DMA,
    ]

    def body(all_h, sem_h, c_h, uid_h, pid_h, nid_h,
             oeu, opos, oneg, ocu, oz, idxb, eub, posb, negb, cub, zb, sem):
        c = lax.axis_index("c")
        s = lax.axis_index("s")
        wid = s * NC + c
        base = wid * rows_per
        pltpu.sync_copy(uid_h.at[pl.ds(base, 128)], idxb.at[16])
        pltpu.sync_copy(pid_h.at[pl.ds(base, 128)], idxb.at[4])
        pltpu.sync_copy(nid_h.at[pl.ds(base, 128)], idxb.at[8])
        for g in range(4):
            for c8 in range(8):
                sl = pl.ds(c8 * L, L)
                u = idxb[16, sl]
                idxb[g, sl] = u + g * N
                idxb[12 + g, sl] = u + g * NU
                if g > 0:
                    idxb[4 + g, sl] = idxb[4, sl] + g * N
                    idxb[8 + g, sl] = idxb[8, sl] + g * N
        ds = []
        for g in range(4):
            ds.append(pltpu.async_copy(all_h.at[idxb.at[g]], eub.at[g], sem))
            ds.append(pltpu.async_copy(all_h.at[idxb.at[4 + g]], posb.at[g],
                                       sem))
            ds.append(pltpu.async_copy(all_h.at[idxb.at[8 + g]], negb.at[g],
                                       sem))
            ds.append(pltpu.async_copy(c_h.at[idxb.at[12 + g]], cub.at[g],
                                       sem))
        ds.append(pltpu.async_copy(sem_h.at[idxb.at[16]], zb, sem))
        for d in ds:
            d.wait()
        for g in range(4):
            pltpu.sync_copy(eub.at[g], oeu.at[g, pl.ds(base, rows_per)])
            pltpu.sync_copy(posb.at[g], opos.at[g, pl.ds(base, rows_per)])
            pltpu.sync_copy(negb.at[g], oneg.at[g, pl.ds(base, rows_per)])
            pltpu.sync_copy(cub.at[g], ocu.at[g, pl.ds(base, rows_per)])
        pltpu.sync_copy(zb, oz.at[pl.ds(base, rows_per)])

    mesh = plsc.VectorSubcoreMesh(**_MESH)
    kern = pl.kernel(
        body,
        out_type=(
            jax.ShapeDtypeStruct((4, B, L), jnp.float32),
            jax.ShapeDtypeStruct((4, B, L), jnp.float32),
            jax.ShapeDtypeStruct((4, B, L), jnp.float32),
            jax.ShapeDtypeStruct((4, B, L), jnp.float32),
            jax.ShapeDtypeStruct((B, SEMD), jnp.float32),
        ),
        mesh=mesh,
        scratch_types=scratch,
        compiler_params=pltpu.CompilerParams(use_tc_tiling_on_sc=False),
        name="batch_gather_sc",
    )
    return kern(all_flat, sem_tab, c_flat, uid, pid, nid)


def _mean3(a, b, c):
    R = a.shape[0]
    blk = 2000

    def body(ar, br, cr, orr):
        orr[...] = (ar[...] + br[...] + cr[...]) * (1.0 / 3.0)

    return pl.pallas_call(
        body,
        grid=(R // blk,),
        in_specs=[pl.BlockSpec((blk, 128), lambda i: (i, 0))] * 3,
        out_specs=pl.BlockSpec((blk, 128), lambda i: (i, 0)),
        out_shape=jax.ShapeDtypeStruct((R, 128), jnp.float32),
    )(a, b, c)


def _tail(eu, pos, neg, cu, z, W_proj, b_proj, W_gate, b_gate):
    BB = 512

    def body(eu_r, pos_r, neg_r, cu_r, z_r, wp_r, bp_r, wg_r, bg_r, o_r):
        e = jnp.concatenate([eu_r[j] for j in range(4)], axis=-1)
        p = jnp.concatenate([pos_r[j] for j in range(4)], axis=-1)
        n = jnp.concatenate([neg_r[j] for j in range(4)], axis=-1)
        cc = jnp.concatenate([cu_r[j] for j in range(4)], axis=-1)
        wg = wg_r[...]
        delta = (jnp.dot(z_r[...], wp_r[...],
                         preferred_element_type=jnp.float32) + bp_r[...])
        h = (jnp.dot(e, wg[0:64], preferred_element_type=jnp.float32)
             + jnp.dot(cc, wg[64:128], preferred_element_type=jnp.float32)
             + jnp.dot(delta, wg[128:192], preferred_element_type=jnp.float32)
             + bg_r[...])
        gate = jax.nn.sigmoid(h)
        ue = e + gate * delta
        ps = jnp.sum(ue * p, axis=1)
        ns = jnp.sum(ue * n, axis=1)
        o_r[0:1, :] = ps.reshape(1, BB)
        o_r[1:2, :] = ns.reshape(1, BB)

    pm = pl.BlockSpec((4, BB, L), lambda i: (0, i, 0))
    return pl.pallas_call(
        body,
        grid=(B // BB,),
        in_specs=[
            pm, pm, pm, pm,
            pl.BlockSpec((BB, SEMD), lambda i: (i, 0)),
            pl.BlockSpec((SEMD, D), lambda i: (0, 0)),
            pl.BlockSpec((1, D), lambda i: (0, 0)),
            pl.BlockSpec((3 * D, D), lambda i: (0, 0)),
            pl.BlockSpec((1, D), lambda i: (0, 0)),
        ],
        out_specs=pl.BlockSpec((2, BB), lambda i: (0, i)),
        out_shape=jax.ShapeDtypeStruct((2, B), jnp.float32),
    )(eu, pos, neg, cu, z, W_proj, b_proj, W_gate, b_gate)


def _pad_edges(dst, src, val, e_pad, n_dst, n_src):
    e = dst.shape[0]
    pad = e_pad - e
    ar = jnp.arange(pad, dtype=jnp.int32)
    dst = jnp.concatenate([dst.astype(jnp.int32), ar % n_dst])
    src = jnp.concatenate([src.astype(jnp.int32), ar % n_src])
    val = jnp.concatenate([val, jnp.zeros((pad,), jnp.float32)])
    return (dst.reshape(e_pad // 128, 128), src.reshape(e_pad // 128, 128),
            val)


def kernel(adj_indices, adj_values, ui_indices, ui_values, user_ids,
           pos_item_ids, neg_item_ids, emb_user, emb_item, sem_table,
           W_proj, b_proj, W_gate, b_gate):
    EA_P = 16 * 74 * CHUNK
    EU_P = 16 * 38 * CHUNK
    dstA, srcA, valA = _pad_edges(adj_indices[0], adj_indices[1], adj_values,
                                  EA_P, N, N)
    e0 = jnp.concatenate([emb_user, emb_item], axis=0)
    e0_pl = e0.reshape(N, 4, L).transpose(1, 0, 2)
    e1_pl = _spmm_planes(dstA, srcA, valA, e0_pl.reshape(4 * N, L),
                         N, 74, 800, N, False)
    e2_pl = _spmm_planes(dstA, srcA, valA, e1_pl.reshape(4 * N, L),
                         N, 74, 800, N, False)
    all_pl = _mean3(e0_pl.reshape(-1, 128), e1_pl.reshape(-1, 128),
                    e2_pl.reshape(-1, 128)).reshape(4 * N, L)

    dstU, srcU, valU = _pad_edges(ui_indices[0],
                                  ui_indices[1].astype(jnp.int32) + NU,
                                  ui_values, EU_P, NU, N)
    c_pl = _spmm_planes(dstU, srcU, valU, all_pl, NU, 38, 400, N, True)

    uid = user_ids.astype(jnp.int32)
    pid = pos_item_ids.astype(jnp.int32) + NU
    nid = neg_item_ids.astype(jnp.int32) + NU
    eu, pos, neg, cu, z = _batch_gather(all_pl, sem_table,
                                        c_pl.reshape(4 * NU, L),
                                        uid, pid, nid)

    return _tail(eu, pos, neg, cu, z, W_proj, b_proj.reshape(1, D),
                 W_gate, b_gate.reshape(1, D))


# speedup vs baseline: 8.3555x; 1.2933x over previous
import functools

import jax
import jax.numpy as jnp
from jax import lax
from jax.experimental import pallas as pl
from jax.experimental.pallas import tpu as pltpu
from jax.experimental.pallas import tpu_sc as plsc

NU = 50000
NI = 50000
N = NU + NI
D = 64
SEMD = 256
B = 4096
NC = 2
NS = 16
L = 16

CHUNK = 1024
KSUB = CHUNK // 128

_MESH = dict(core_axis_name="c", subcore_axis_name="s", num_cores=NC,
             num_subcores=NS)


def _splat(vec, idx):
    dn = lax.GatherDimensionNumbers(offset_dims=(), collapsed_slice_dims=(0,),
                                    start_index_map=(0,))
    return lax.gather(vec, idx[:, None], dn, slice_sizes=(1,),
                      mode=lax.GatherScatterMode.PROMISE_IN_BOUNDS)


def _spmm_planes(pk2, val1, table_flat, n_rows, nsuper, zch, plane_rows,
                 normalize):
    n_zch = n_rows // zch
    wb_iters = (n_zch + NS - 1) // NS
    SUP = 2048
    SUB = 512
    ept = nsuper * SUP

    scratch = [
        pltpu.VMEM_SHARED((n_rows, L), jnp.float32),
        pltpu.VMEM((SUB, L), jnp.float32),
        pltpu.VMEM((SUB, L), jnp.float32),
        pltpu.VMEM((16, 2, 128), jnp.int32),
        pltpu.VMEM((2048,), jnp.float32),
        pltpu.VMEM((16, 128), jnp.int32),
        pltpu.SemaphoreType.DMA,
        pltpu.SemaphoreType.DMA,
    ]
    if normalize:
        scratch += [
            pltpu.VMEM_SHARED((n_rows,), jnp.float32),
            pltpu.VMEM((zch,), jnp.float32),
        ]

    def body(pk_h, val_h, tab_h, out_h, acc, rows0, rows1, eb, valb, gx,
             gsem, ssem, *extra):
        c = lax.axis_index("c")
        s_ = lax.axis_index("s")
        iota = lax.iota(jnp.int32, L)
        cols = [iota * 0 + j for j in range(L)]
        rowsl = [rows0, rows1]
        if normalize:
            rsacc, rsbuf = extra

        for gl in range(2):
            g = c * 2 + gl
            off = g * plane_rows

            def zrow(i, _):
                rows0[i, :] = jnp.zeros((L,), jnp.float32)
                return 0
            lax.fori_loop(0, zch, zrow, 0)
            if normalize and gl == 0:
                def zrs(i, _):
                    rsbuf[pl.ds(i * L, L)] = jnp.zeros((L,), jnp.float32)
                    return 0
                lax.fori_loop(0, zch // L, zrs, 0)
            for k in range(wb_iters):
                idx = s_ + NS * k

                @pl.when(idx < n_zch)
                def _():
                    pltpu.sync_copy(rows0.at[pl.ds(0, zch)],
                                    acc.at[pl.ds(idx * zch, zch)])
                    if normalize and gl == 0:
                        pltpu.sync_copy(rsbuf, rsacc.at[pl.ds(idx * zch, zch)])
            plsc.subcore_barrier()

            def fire_gather(k):
                rb = [pltpu.async_copy(tab_h.at[gx.at[4 * k + j]],
                                       rowsl[k % 2].at[pl.ds(j * 128, 128)],
                                       gsem)
                      for j in range(4)]
                return rb

            def fire_scatter(k):
                ds_ = [pltpu.async_copy(rowsl[k % 2].at[pl.ds(j * 128, 128)],
                                        acc.at[eb.at[4 * k + j, 1]], ssem,
                                        add=True)
                       for j in range(4)]
                if normalize and gl == 0:
                    ds_ += [pltpu.async_copy(
                        valb.at[pl.ds((4 * k + j) * 128, 128)],
                        rsacc.at[eb.at[4 * k + j, 1]], ssem, add=True)
                            for j in range(4)]
                return ds_

            def scale(k):
                for j in range(4):
                    def sc8(c8, _):
                        rbase = j * 128 + c8 * L
                        vv = valb[pl.ds(k * 512 + rbase, L)]
                        rw = rowsl[k % 2]
                        for jj in range(L):
                            sp = _splat(vv, cols[jj])
                            rw[rbase + jj, :] = rw[rbase + jj, :] * sp
                        return 0
                    lax.fori_loop(0, 8, sc8, 0)

            def sup_body(u, _):
                rb = s_ * (ept // 128) + u * 16
                pltpu.sync_copy(pk_h.at[pl.ds(rb, 16)], eb)
                pltpu.sync_copy(val_h.at[pl.ds(s_ * ept + u * 2048, 2048)],
                                valb)
                for r in range(16):
                    for c8 in range(8):
                        sl = pl.ds(c8 * L, L)
                        gx[r, sl] = eb[r, 0, sl] + off
                gd = fire_gather(0)
                sd = []
                for k in range(4):
                    for d in gd:
                        d.wait()
                    for d in sd:
                        d.wait()
                    if k < 3:
                        gd = fire_gather(k + 1)
                    scale(k)
                    sd = fire_scatter(k)
                for d in sd:
                    d.wait()
                return 0
            lax.fori_loop(0, nsuper, sup_body, 0)
            plsc.subcore_barrier()

            for k in range(wb_iters):
                idx = s_ + NS * k

                @pl.when(idx < n_zch)
                def _():
                    if normalize:
                        pltpu.sync_copy(acc.at[pl.ds(idx * zch, zch)],
                                        rows0.at[pl.ds(0, zch)])
                        pltpu.sync_copy(rsacc.at[pl.ds(idx * zch, zch)], rsbuf)

                        def dv(b2, _):
                            rb16 = b2 * L
                            rsv = 1.0 / jnp.maximum(rsbuf[pl.ds(rb16, L)],
                                                    1.0)
                            for j in range(L):
                                sp = _splat(rsv, cols[j])
                                rows0[rb16 + j, :] = rows0[rb16 + j, :] * sp
                            return 0
                        lax.fori_loop(0, zch // L, dv, 0)
                        pltpu.sync_copy(rows0.at[pl.ds(0, zch)],
                                        out_h.at[g, pl.ds(idx * zch, zch)])
                    else:
                        pltpu.sync_copy(acc.at[pl.ds(idx * zch, zch)],
                                        out_h.at[g, pl.ds(idx * zch, zch)])
            plsc.subcore_barrier()

    mesh = plsc.VectorSubcoreMesh(**_MESH)
    kern = pl.kernel(
        body,
        out_type=jax.ShapeDtypeStruct((4, n_rows, L), jnp.float32),
        mesh=mesh,
        scratch_types=scratch,
        compiler_params=pltpu.CompilerParams(use_tc_tiling_on_sc=False),
        name=f"spmm_sc_{n_rows}_{nsuper}",
    )
    return kern(pk2, val1, table_flat)


def _batch_gather(all_flat, sem_tab, c_flat, uid, pid, nid):
    rows_per = B // (NC * NS)

    scratch = [
        pltpu.VMEM((17, 128), jnp.int32),
        pltpu.VMEM((4, rows_per, L), jnp.float32),
        pltpu.VMEM((4, rows_per, L), jnp.float32),
        pltpu.VMEM((4, rows_per, L), jnp.float32),
        pltpu.VMEM((4, rows_per, L), jnp.float32),
        pltpu.VMEM((rows_per, SEMD), jnp.float32),
        pltpu.SemaphoreType.DMA,
    ]

    def body(all_h, sem_h, c_h, uid_h, pid_h, nid_h,
             oeu, opos, oneg, ocu, oz, idxb, eub, posb, negb, cub, zb, sem):
        c = lax.axis_index("c")
        s = lax.axis_index("s")
        wid = s * NC + c
        base = wid * rows_per
        pltpu.sync_copy(uid_h.at[pl.ds(base, 128)], idxb.at[16])
        pltpu.sync_copy(pid_h.at[pl.ds(base, 128)], idxb.at[4])
        pltpu.sync_copy(nid_h.at[pl.ds(base, 128)], idxb.at[8])
        for g in range(4):
            for c8 in range(8):
                sl = pl.ds(c8 * L, L)
                u = idxb[16, sl]
                idxb[g, sl] = u + g * N
                idxb[12 + g, sl] = u + g * NU
                if g > 0:
                    idxb[4 + g, sl] = idxb[4, sl] + g * N
                    idxb[8 + g, sl] = idxb[8, sl] + g * N
        ds = []
        for g in range(4):
            ds.append(pltpu.async_copy(all_h.at[idxb.at[g]], eub.at[g], sem))
            ds.append(pltpu.async_copy(all_h.at[idxb.at[4 + g]], posb.at[g],
                                       sem))
            ds.append(pltpu.async_copy(all_h.at[idxb.at[8 + g]], negb.at[g],
                                       sem))
            ds.append(pltpu.async_copy(c_h.at[idxb.at[12 + g]], cub.at[g],
                                       sem))
        ds.append(pltpu.async_copy(sem_h.at[idxb.at[16]], zb, sem))
        for d in ds:
            d.wait()
        for g in range(4):
            pltpu.sync_copy(eub.at[g], oeu.at[g, pl.ds(base, rows_per)])
            pltpu.sync_copy(posb.at[g], opos.at[g, pl.ds(base, rows_per)])
            pltpu.sync_copy(negb.at[g], oneg.at[g, pl.ds(base, rows_per)])
            pltpu.sync_copy(cub.at[g], ocu.at[g, pl.ds(base, rows_per)])
        pltpu.sync_copy(zb, oz.at[pl.ds(base, rows_per)])

    mesh = plsc.VectorSubcoreMesh(**_MESH)
    kern = pl.kernel(
        body,
        out_type=(
            jax.ShapeDtypeStruct((4, B, L), jnp.float32),
            jax.ShapeDtypeStruct((4, B, L), jnp.float32),
            jax.ShapeDtypeStruct((4, B, L), jnp.float32),
            jax.ShapeDtypeStruct((4, B, L), jnp.float32),
            jax.ShapeDtypeStruct((B, SEMD), jnp.float32),
        ),
        mesh=mesh,
        scratch_types=scratch,
        compiler_params=pltpu.CompilerParams(use_tc_tiling_on_sc=False),
        name="batch_gather_sc",
    )
    return kern(all_flat, sem_tab, c_flat, uid, pid, nid)


def _mean3(a, b, c):
    R = a.shape[0]
    blk = 2000

    def body(ar, br, cr, orr):
        orr[...] = (ar[...] + br[...] + cr[...]) * (1.0 / 3.0)

    return pl.pallas_call(
        body,
        grid=(R // blk,),
        in_specs=[pl.BlockSpec((blk, 128), lambda i: (i, 0))] * 3,
        out_specs=pl.BlockSpec((blk, 128), lambda i: (i, 0)),
        out_shape=jax.ShapeDtypeStruct((R, 128), jnp.float32),
    )(a, b, c)


def _tail(eu, pos, neg, cu, z, W_proj, b_proj, W_gate, b_gate):
    BB = 512

    def body(eu_r, pos_r, neg_r, cu_r, z_r, wp_r, bp_r, wg_r, bg_r, o_r):
        e = jnp.concatenate([eu_r[j] for j in range(4)], axis=-1)
        p = jnp.concatenate([pos_r[j] for j in range(4)], axis=-1)
        n = jnp.concatenate([neg_r[j] for j in range(4)], axis=-1)
        cc = jnp.concatenate([cu_r[j] for j in range(4)], axis=-1)
        wg = wg_r[...]
        delta = (jnp.dot(z_r[...], wp_r[...],
                         preferred_element_type=jnp.float32) + bp_r[...])
        h = (jnp.dot(e, wg[0:64], preferred_element_type=jnp.float32)
             + jnp.dot(cc, wg[64:128], preferred_element_type=jnp.float32)
             + jnp.dot(delta, wg[128:192], preferred_element_type=jnp.float32)
             + bg_r[...])
        gate = jax.nn.sigmoid(h)
        ue = e + gate * delta
        ps = jnp.sum(ue * p, axis=1)
        ns = jnp.sum(ue * n, axis=1)
        o_r[0:1, :] = ps.reshape(1, BB)
        o_r[1:2, :] = ns.reshape(1, BB)

    pm = pl.BlockSpec((4, BB, L), lambda i: (0, i, 0))
    return pl.pallas_call(
        body,
        grid=(B // BB,),
        in_specs=[
            pm, pm, pm, pm,
            pl.BlockSpec((BB, SEMD), lambda i: (i, 0)),
            pl.BlockSpec((SEMD, D), lambda i: (0, 0)),
            pl.BlockSpec((1, D), lambda i: (0, 0)),
            pl.BlockSpec((3 * D, D), lambda i: (0, 0)),
            pl.BlockSpec((1, D), lambda i: (0, 0)),
        ],
        out_specs=pl.BlockSpec((2, BB), lambda i: (0, i)),
        out_shape=jax.ShapeDtypeStruct((2, B), jnp.float32),
    )(eu, pos, neg, cu, z, W_proj, b_proj, W_gate, b_gate)


def _pad_edges(dst, src, val, e_pad, n_dst, n_src):
    e = dst.shape[0]
    pad = e_pad - e
    ar = jnp.arange(pad, dtype=jnp.int32)
    dst = jnp.concatenate([dst.astype(jnp.int32), ar % n_dst])
    src = jnp.concatenate([src.astype(jnp.int32), ar % n_src])
    val = jnp.concatenate([val, jnp.zeros((pad,), jnp.float32)])
    return jnp.stack([src.reshape(e_pad // 128, 128),
                      dst.reshape(e_pad // 128, 128)], axis=1), val


def kernel(adj_indices, adj_values, ui_indices, ui_values, user_ids,
           pos_item_ids, neg_item_ids, emb_user, emb_item, sem_table,
           W_proj, b_proj, W_gate, b_gate):
    EA_P = 16 * 37 * 2048
    EU_P = 16 * 19 * 2048
    pkA, valA = _pad_edges(adj_indices[0], adj_indices[1], adj_values,
                           EA_P, N, N)
    e0 = jnp.concatenate([emb_user, emb_item], axis=0)
    e0_pl = e0.reshape(N, 4, L).transpose(1, 0, 2)
    e1_pl = _spmm_planes(pkA, valA, e0_pl.reshape(4 * N, L), N, 37, 400, N,
                         False)
    e2_pl = _spmm_planes(pkA, valA, e1_pl.reshape(4 * N, L), N, 37, 400, N,
                         False)
    all_pl = _mean3(e0_pl.reshape(-1, 128), e1_pl.reshape(-1, 128),
                    e2_pl.reshape(-1, 128)).reshape(4 * N, L)

    pkU, valU = _pad_edges(ui_indices[0],
                           ui_indices[1].astype(jnp.int32) + NU,
                           ui_values, EU_P, NU, N)
    c_pl = _spmm_planes(pkU, valU, all_pl, NU, 19, 400, N, True)

    uid = user_ids.astype(jnp.int32)
    pid = pos_item_ids.astype(jnp.int32) + NU
    nid = neg_item_ids.astype(jnp.int32) + NU
    eu, pos, neg, cu, z = _batch_gather(all_pl, sem_table,
                                        c_pl.reshape(4 * NU, L),
                                        uid, pid, nid)

    return _tail(eu, pos, neg, cu, z, W_proj, b_proj.reshape(1, D),
                 W_gate, b_gate.reshape(1, D))


# speedup vs baseline: 8.8321x; 1.0570x over previous
import functools

import jax
import jax.numpy as jnp
from jax import lax
from jax.experimental import pallas as pl
from jax.experimental.pallas import tpu as pltpu
from jax.experimental.pallas import tpu_sc as plsc

NU = 50000
NI = 50000
N = NU + NI
D = 64
SEMD = 256
B = 4096
NC = 2
NS = 16
L = 16

CHUNK = 1024
KSUB = CHUNK // 128

_MESH = dict(core_axis_name="c", subcore_axis_name="s", num_cores=NC,
             num_subcores=NS)


def _splat(vec, idx):
    dn = lax.GatherDimensionNumbers(offset_dims=(), collapsed_slice_dims=(0,),
                                    start_index_map=(0,))
    return lax.gather(vec, idx[:, None], dn, slice_sizes=(1,),
                      mode=lax.GatherScatterMode.PROMISE_IN_BOUNDS)


def _spmm_planes(pk2, val1, table_flat, n_rows, nsuper, zch, plane_rows,
                 normalize):
    n_zch = n_rows // zch
    wb_iters = (n_zch + NS - 1) // NS
    SUP = 512
    KS = SUP // 128
    ept = nsuper * SUP
    npairs = nsuper // 2

    scratch = [
        pltpu.VMEM_SHARED((n_rows, L), jnp.float32),
        pltpu.VMEM((SUP, L), jnp.float32),
        pltpu.VMEM((SUP, L), jnp.float32),
        pltpu.VMEM((4, 2, 128), jnp.int32),
        pltpu.VMEM((4, 2, 128), jnp.int32),
        pltpu.VMEM((SUP,), jnp.float32),
        pltpu.VMEM((SUP,), jnp.float32),
        pltpu.VMEM((4, 128), jnp.int32),
        pltpu.VMEM((4, 128), jnp.int32),
        pltpu.SemaphoreType.DMA,
        pltpu.SemaphoreType.DMA,
        pltpu.SemaphoreType.DMA,
    ]
    if normalize:
        scratch += [
            pltpu.VMEM_SHARED((n_rows,), jnp.float32),
            pltpu.VMEM((zch,), jnp.float32),
        ]

    def body(pk_h, val_h, tab_h, out_h, acc, rows0, rows1, eb0, eb1, vb0,
             vb1, gx0, gx1, stsem, gsem, ssem, *extra):
        c = lax.axis_index("c")
        s_ = lax.axis_index("s")
        iota = lax.iota(jnp.int32, L)
        cols = [iota * 0 + j for j in range(L)]
        rowsl = [rows0, rows1]
        if normalize:
            rsacc, rsbuf = extra

        for gl in range(2):
            g = c * 2 + gl
            off = g * plane_rows

            def zrow(i, _):
                rows0[i, :] = jnp.zeros((L,), jnp.float32)
                return 0
            lax.fori_loop(0, zch, zrow, 0)
            if normalize and gl == 0:
                def zrs(i, _):
                    rsbuf[pl.ds(i * L, L)] = jnp.zeros((L,), jnp.float32)
                    return 0
                lax.fori_loop(0, zch // L, zrs, 0)
            for k in range(wb_iters):
                idx = s_ + NS * k

                @pl.when(idx < n_zch)
                def _():
                    pltpu.sync_copy(rows0.at[pl.ds(0, zch)],
                                    acc.at[pl.ds(idx * zch, zch)])
                    if normalize and gl == 0:
                        pltpu.sync_copy(rsbuf, rsacc.at[pl.ds(idx * zch, zch)])
            plsc.subcore_barrier()

            def stage_fire(i, ebX, vbX):
                rb = s_ * (ept // 128) + i * KS
                pltpu.async_copy(pk_h.at[pl.ds(rb, KS)], ebX, stsem)
                pltpu.async_copy(val_h.at[pl.ds(s_ * ept + i * SUP, SUP)],
                                 vbX, stsem)

            def stage_drain(i, ebX, vbX):
                rb = s_ * (ept // 128) + i * KS
                pltpu.make_async_copy(pk_h.at[pl.ds(rb, KS)], ebX,
                                      stsem).wait()
                pltpu.make_async_copy(
                    val_h.at[pl.ds(s_ * ept + i * SUP, SUP)], vbX,
                    stsem).wait()

            def gidx_compute(ebX, gxX):
                for r in range(KS):
                    for c8 in range(8):
                        sl = pl.ds(c8 * L, L)
                        gxX[r, sl] = ebX[r, 0, sl] + off

            def gath_fire(gxX, rowsX):
                for j in range(KS):
                    pltpu.async_copy(tab_h.at[gxX.at[j]],
                                     rowsX.at[pl.ds(j * 128, 128)], gsem)

            def gath_drain(gxX, rowsX):
                for j in range(KS):
                    pltpu.make_async_copy(tab_h.at[gxX.at[j]],
                                          rowsX.at[pl.ds(j * 128, 128)],
                                          gsem).wait()

            def scat_fire(ebX, vbX, rowsX):
                for j in range(KS):
                    pltpu.async_copy(rowsX.at[pl.ds(j * 128, 128)],
                                     acc.at[ebX.at[j, 1]], ssem, add=True)
                if normalize and gl == 0:
                    for j in range(KS):
                        pltpu.async_copy(vbX.at[pl.ds(j * 128, 128)],
                                         rsacc.at[ebX.at[j, 1]], ssem,
                                         add=True)

            def scat_drain(ebX, vbX, rowsX):
                for j in range(KS):
                    pltpu.make_async_copy(rowsX.at[pl.ds(j * 128, 128)],
                                          acc.at[ebX.at[j, 1]], ssem).wait()
                if normalize and gl == 0:
                    for j in range(KS):
                        pltpu.make_async_copy(vbX.at[pl.ds(j * 128, 128)],
                                              rsacc.at[ebX.at[j, 1]],
                                              ssem).wait()

            def scale(vbX, rowsX):
                def sc16(b2, _):
                    rb16 = b2 * L
                    vv = vbX[pl.ds(rb16, L)]
                    for jj in range(L):
                        sp = _splat(vv, cols[jj])
                        rowsX[rb16 + jj, :] = rowsX[rb16 + jj, :] * sp
                    return 0
                lax.fori_loop(0, SUP // L, sc16, 0)

            stage_fire(0, eb0, vb0)
            stage_drain(0, eb0, vb0)
            gidx_compute(eb0, gx0)
            gath_fire(gx0, rows0)

            def pair(t, _):
                i0 = 2 * t
                i1 = i0 + 1
                @pl.when(t > 0)
                def _():
                    scat_drain(eb1, vb1, rows1)
                stage_fire(i1, eb1, vb1)
                gath_drain(gx0, rows0)
                stage_drain(i1, eb1, vb1)
                gidx_compute(eb1, gx1)
                gath_fire(gx1, rows1)
                scale(vb0, rows0)
                scat_fire(eb0, vb0, rows0)
                gath_drain(gx1, rows1)
                scat_drain(eb0, vb0, rows0)

                @pl.when(t < npairs - 1)
                def _():
                    stage_fire(i0 + 2, eb0, vb0)
                    stage_drain(i0 + 2, eb0, vb0)
                    gidx_compute(eb0, gx0)
                    gath_fire(gx0, rows0)
                scale(vb1, rows1)
                scat_fire(eb1, vb1, rows1)
                return 0
            lax.fori_loop(0, npairs, pair, 0)
            scat_drain(eb1, vb1, rows1)
            plsc.subcore_barrier()

            for k in range(wb_iters):
                idx = s_ + NS * k

                @pl.when(idx < n_zch)
                def _():
                    if normalize:
                        pltpu.sync_copy(acc.at[pl.ds(idx * zch, zch)],
                                        rows0.at[pl.ds(0, zch)])
                        pltpu.sync_copy(rsacc.at[pl.ds(idx * zch, zch)], rsbuf)

                        def dv(b2, _):
                            rb16 = b2 * L
                            rsv = 1.0 / jnp.maximum(rsbuf[pl.ds(rb16, L)],
                                                    1.0)
                            for j in range(L):
                                sp = _splat(rsv, cols[j])
                                rows0[rb16 + j, :] = rows0[rb16 + j, :] * sp
                            return 0
                        lax.fori_loop(0, zch // L, dv, 0)
                        pltpu.sync_copy(rows0.at[pl.ds(0, zch)],
                                        out_h.at[g, pl.ds(idx * zch, zch)])
                    else:
                        pltpu.sync_copy(acc.at[pl.ds(idx * zch, zch)],
                                        out_h.at[g, pl.ds(idx * zch, zch)])
            plsc.subcore_barrier()

    mesh = plsc.VectorSubcoreMesh(**_MESH)
    kern = pl.kernel(
        body,
        out_type=jax.ShapeDtypeStruct((4, n_rows, L), jnp.float32),
        mesh=mesh,
        scratch_types=scratch,
        compiler_params=pltpu.CompilerParams(use_tc_tiling_on_sc=False),
        name=f"spmm_sc_{n_rows}_{nsuper}",
    )
    return kern(pk2, val1, table_flat)


def _batch_gather(all_flat, sem_tab, c_flat, uid, pid, nid):
    rows_per = B // (NC * NS)

    scratch = [
        pltpu.VMEM((17, 128), jnp.int32),
        pltpu.VMEM((4, rows_per, L), jnp.float32),
        pltpu.VMEM((4, rows_per, L), jnp.float32),
        pltpu.VMEM((4, rows_per, L), jnp.float32),
        pltpu.VMEM((4, rows_per, L), jnp.float32),
        pltpu.VMEM((rows_per, SEMD), jnp.float32),
        pltpu.SemaphoreType.DMA,
    ]

    def body(all_h, sem_h, c_h, uid_h, pid_h, nid_h,
             oeu, opos, oneg, ocu, oz, idxb, eub, posb, negb, cub, zb, sem):
        c = lax.axis_index("c")
        s = lax.axis_index("s")
        wid = s * NC + c
        base = wid * rows_per
        pltpu.sync_copy(uid_h.at[pl.ds(base, 128)], idxb.at[16])
        pltpu.sync_copy(pid_h.at[pl.ds(base, 128)], idxb.at[4])
        pltpu.sync_copy(nid_h.at[pl.ds(base, 128)], idxb.at[8])
        for g in range(4):
            for c8 in range(8):
                sl = pl.ds(c8 * L, L)
                u = idxb[16, sl]
                idxb[g, sl] = u + g * N
                idxb[12 + g, sl] = u + g * NU
                if g > 0:
                    idxb[4 + g, sl] = idxb[4, sl] + g * N
                    idxb[8 + g, sl] = idxb[8, sl] + g * N
        ds = []
        for g in range(4):
            ds.append(pltpu.async_copy(all_h.at[idxb.at[g]], eub.at[g], sem))
            ds.append(pltpu.async_copy(all_h.at[idxb.at[4 + g]], posb.at[g],
                                       sem))
            ds.append(pltpu.async_copy(all_h.at[idxb.at[8 + g]], negb.at[g],
                                       sem))
            ds.append(pltpu.async_copy(c_h.at[idxb.at[12 + g]], cub.at[g],
                                       sem))
        ds.append(pltpu.async_copy(sem_h.at[idxb.at[16]], zb, sem))
        for d in ds:
            d.wait()
        for g in range(4):
            pltpu.sync_copy(eub.at[g], oeu.at[g, pl.ds(base, rows_per)])
            pltpu.sync_copy(posb.at[g], opos.at[g, pl.ds(base, rows_per)])
            pltpu.sync_copy(negb.at[g], oneg.at[g, pl.ds(base, rows_per)])
            pltpu.sync_copy(cub.at[g], ocu.at[g, pl.ds(base, rows_per)])
        pltpu.sync_copy(zb, oz.at[pl.ds(base, rows_per)])

    mesh = plsc.VectorSubcoreMesh(**_MESH)
    kern = pl.kernel(
        body,
        out_type=(
            jax.ShapeDtypeStruct((4, B, L), jnp.float32),
            jax.ShapeDtypeStruct((4, B, L), jnp.float32),
            jax.ShapeDtypeStruct((4, B, L), jnp.float32),
            jax.ShapeDtypeStruct((4, B, L), jnp.float32),
            jax.ShapeDtypeStruct((B, SEMD), jnp.float32),
        ),
        mesh=mesh,
        scratch_types=scratch,
        compiler_params=pltpu.CompilerParams(use_tc_tiling_on_sc=False),
        name="batch_gather_sc",
    )
    return kern(all_flat, sem_tab, c_flat, uid, pid, nid)


def _mean3(a, b, c):
    R = a.shape[0]
    blk = 2000

    def body(ar, br, cr, orr):
        orr[...] = (ar[...] + br[...] + cr[...]) * (1.0 / 3.0)

    return pl.pallas_call(
        body,
        grid=(R // blk,),
        in_specs=[pl.BlockSpec((blk, 128), lambda i: (i, 0))] * 3,
        out_specs=pl.BlockSpec((blk, 128), lambda i: (i, 0)),
        out_shape=jax.ShapeDtypeStruct((R, 128), jnp.float32),
    )(a, b, c)


def _tail(eu, pos, neg, cu, z, W_proj, b_proj, W_gate, b_gate):
    BB = 512

    def body(eu_r, pos_r, neg_r, cu_r, z_r, wp_r, bp_r, wg_r, bg_r, o_r):
        e = jnp.concatenate([eu_r[j] for j in range(4)], axis=-1)
        p = jnp.concatenate([pos_r[j] for j in range(4)], axis=-1)
        n = jnp.concatenate([neg_r[j] for j in range(4)], axis=-1)
        cc = jnp.concatenate([cu_r[j] for j in range(4)], axis=-1)
        wg = wg_r[...]
        delta = (jnp.dot(z_r[...], wp_r[...],
                         preferred_element_type=jnp.float32) + bp_r[...])
        h = (jnp.dot(e, wg[0:64], preferred_element_type=jnp.float32)
             + jnp.dot(cc, wg[64:128], preferred_element_type=jnp.float32)
             + jnp.dot(delta, wg[128:192], preferred_element_type=jnp.float32)
             + bg_r[...])
        gate = jax.nn.sigmoid(h)
        ue = e + gate * delta
        ps = jnp.sum(ue * p, axis=1)
        ns = jnp.sum(ue * n, axis=1)
        o_r[0:1, :] = ps.reshape(1, BB)
        o_r[1:2, :] = ns.reshape(1, BB)

    pm = pl.BlockSpec((4, BB, L), lambda i: (0, i, 0))
    return pl.pallas_call(
        body,
        grid=(B // BB,),
        in_specs=[
            pm, pm, pm, pm,
            pl.BlockSpec((BB, SEMD), lambda i: (i, 0)),
            pl.BlockSpec((SEMD, D), lambda i: (0, 0)),
            pl.BlockSpec((1, D), lambda i: (0, 0)),
            pl.BlockSpec((3 * D, D), lambda i: (0, 0)),
            pl.BlockSpec((1, D), lambda i: (0, 0)),
        ],
        out_specs=pl.BlockSpec((2, BB), lambda i: (0, i)),
        out_shape=jax.ShapeDtypeStruct((2, B), jnp.float32),
    )(eu, pos, neg, cu, z, W_proj, b_proj, W_gate, b_gate)


def _pad_edges(dst, src, val, e_pad, n_dst, n_src):
    e = dst.shape[0]
    pad = e_pad - e
    ar = jnp.arange(pad, dtype=jnp.int32)
    dst = jnp.concatenate([dst.astype(jnp.int32), ar % n_dst])
    src = jnp.concatenate([src.astype(jnp.int32), ar % n_src])
    val = jnp.concatenate([val, jnp.zeros((pad,), jnp.float32)])
    return jnp.stack([src.reshape(e_pad // 128, 128),
                      dst.reshape(e_pad // 128, 128)], axis=1), val


def kernel(adj_indices, adj_values, ui_indices, ui_values, user_ids,
           pos_item_ids, neg_item_ids, emb_user, emb_item, sem_table,
           W_proj, b_proj, W_gate, b_gate):
    EA_P = 16 * 148 * 512
    EU_P = 16 * 76 * 512
    pkA, valA = _pad_edges(adj_indices[0], adj_indices[1], adj_values,
                           EA_P, N, N)
    e0 = jnp.concatenate([emb_user, emb_item], axis=0)
    e0_pl = e0.reshape(N, 4, L).transpose(1, 0, 2)
    e1_pl = _spmm_planes(pkA, valA, e0_pl.reshape(4 * N, L), N, 148, 400, N,
                         False)
    e2_pl = _spmm_planes(pkA, valA, e1_pl.reshape(4 * N, L), N, 148, 400, N,
                         False)
    all_pl = _mean3(e0_pl.reshape(-1, 128), e1_pl.reshape(-1, 128),
                    e2_pl.reshape(-1, 128)).reshape(4 * N, L)

    pkU, valU = _pad_edges(ui_indices[0],
                           ui_indices[1].astype(jnp.int32) + NU,
                           ui_values, EU_P, NU, N)
    c_pl = _spmm_planes(pkU, valU, all_pl, NU, 76, 400, N, True)

    uid = user_ids.astype(jnp.int32)
    pid = pos_item_ids.astype(jnp.int32) + NU
    nid = neg_item_ids.astype(jnp.int32) + NU
    eu, pos, neg, cu, z = _batch_gather(all_pl, sem_table,
                                        c_pl.reshape(4 * NU, L),
                                        uid, pid, nid)

    return _tail(eu, pos, neg, cu, z, W_proj, b_proj.reshape(1, D),
                 W_gate, b_gate.reshape(1, D))


# speedup vs baseline: 8.9110x; 1.0089x over previous
import functools

import jax
import jax.numpy as jnp
from jax import lax
from jax.experimental import pallas as pl
from jax.experimental.pallas import tpu as pltpu
from jax.experimental.pallas import tpu_sc as plsc

NU = 50000
NI = 50000
N = NU + NI
D = 64
SEMD = 256
B = 4096
NC = 2
NS = 16
L = 16

CHUNK = 1024
KSUB = CHUNK // 128

_MESH = dict(core_axis_name="c", subcore_axis_name="s", num_cores=NC,
             num_subcores=NS)


def _splat(vec, idx):
    dn = lax.GatherDimensionNumbers(offset_dims=(), collapsed_slice_dims=(0,),
                                    start_index_map=(0,))
    return lax.gather(vec, idx[:, None], dn, slice_sizes=(1,),
                      mode=lax.GatherScatterMode.PROMISE_IN_BOUNDS)


def _spmm_planes(pk2, val1, table_flat, n_rows, nsuper, zch, plane_rows,
                 normalize, batch=None):
    n_zch = n_rows // zch
    wb_iters = (n_zch + NS - 1) // NS
    SUP = 512
    KS = SUP // 128
    ept = nsuper * SUP
    npairs = nsuper // 2

    scratch = [
        pltpu.VMEM_SHARED((n_rows, L), jnp.float32),
        pltpu.VMEM((SUP, L), jnp.float32),
        pltpu.VMEM((SUP, L), jnp.float32),
        pltpu.VMEM((4, 2, 128), jnp.int32),
        pltpu.VMEM((4, 2, 128), jnp.int32),
        pltpu.VMEM((SUP,), jnp.float32),
        pltpu.VMEM((SUP,), jnp.float32),
        pltpu.VMEM((4, 128), jnp.int32),
        pltpu.VMEM((4, 128), jnp.int32),
        pltpu.SemaphoreType.DMA,
        pltpu.SemaphoreType.DMA,
        pltpu.SemaphoreType.DMA,
    ]
    if normalize:
        scratch += [
            pltpu.VMEM_SHARED((n_rows,), jnp.float32),
            pltpu.VMEM((zch,), jnp.float32),
        ]
    if batch is not None:
        scratch += [
            pltpu.VMEM((4, 128), jnp.int32),
            pltpu.VMEM((128, SEMD), jnp.float32),
        ]

    def body(*refs):
        it = iter(refs)
        pk_h, val_h, tab_h = next(it), next(it), next(it)
        if batch is not None:
            sem_h, uid_h, pid_h, nid_h = next(it), next(it), next(it), next(it)
            oeu, opos, oneg, ocu, oz = (next(it), next(it), next(it),
                                        next(it), next(it))
        else:
            out_h = next(it)
        acc, rows0, rows1 = next(it), next(it), next(it)
        eb0, eb1, vb0, vb1, gx0, gx1 = (next(it), next(it), next(it),
                                        next(it), next(it), next(it))
        stsem, gsem, ssem = next(it), next(it), next(it)
        if normalize:
            rsacc, rsbuf = next(it), next(it)
        if batch is not None:
            bidx, zb = next(it), next(it)
        c = lax.axis_index("c")
        s_ = lax.axis_index("s")
        iota = lax.iota(jnp.int32, L)
        cols = [iota * 0 + j for j in range(L)]

        if batch is not None:
            wid = s_ * NC + c
            wbase = wid * 128
            pltpu.sync_copy(uid_h.at[pl.ds(wbase, 128)], bidx.at[0])
            pltpu.sync_copy(pid_h.at[pl.ds(wbase, 128)], bidx.at[1])
            pltpu.sync_copy(nid_h.at[pl.ds(wbase, 128)], bidx.at[2])
            dz = pltpu.async_copy(sem_h.at[bidx.at[0]], zb, gsem)
            for tix, obuf in ((0, oeu), (1, opos), (2, oneg)):
                for g4 in range(4):
                    for c8 in range(8):
                        sl = pl.ds(c8 * L, L)
                        gx0[g4, sl] = bidx[tix, sl] + g4 * plane_rows
                for g4 in range(4):
                    pltpu.sync_copy(tab_h.at[gx0.at[g4]],
                                    rows0.at[pl.ds(g4 * 128, 128)])
                for g4 in range(4):
                    pltpu.sync_copy(rows0.at[pl.ds(g4 * 128, 128)],
                                    obuf.at[g4, pl.ds(wbase, 128)])
            dz.wait()
            pltpu.sync_copy(zb, oz.at[pl.ds(wbase, 128)])

        for gl in range(2):
            g = c * 2 + gl
            off = g * plane_rows

            def zrow(i, _):
                rows0[i, :] = jnp.zeros((L,), jnp.float32)
                return 0
            lax.fori_loop(0, zch, zrow, 0)
            if normalize and gl == 0:
                def zrs(i, _):
                    rsbuf[pl.ds(i * L, L)] = jnp.zeros((L,), jnp.float32)
                    return 0
                lax.fori_loop(0, zch // L, zrs, 0)
            for k in range(wb_iters):
                idx = s_ + NS * k

                @pl.when(idx < n_zch)
                def _():
                    pltpu.sync_copy(rows0.at[pl.ds(0, zch)],
                                    acc.at[pl.ds(idx * zch, zch)])
                    if normalize and gl == 0:
                        pltpu.sync_copy(rsbuf, rsacc.at[pl.ds(idx * zch, zch)])
            plsc.subcore_barrier()

            def stage_fire(i, ebX, vbX):
                rb = s_ * (ept // 128) + i * KS
                pltpu.async_copy(pk_h.at[pl.ds(rb, KS)], ebX, stsem)
                pltpu.async_copy(val_h.at[pl.ds(s_ * ept + i * SUP, SUP)],
                                 vbX, stsem)

            def stage_drain(i, ebX, vbX):
                rb = s_ * (ept // 128) + i * KS
                pltpu.make_async_copy(pk_h.at[pl.ds(rb, KS)], ebX,
                                      stsem).wait()
                pltpu.make_async_copy(
                    val_h.at[pl.ds(s_ * ept + i * SUP, SUP)], vbX,
                    stsem).wait()

            def gidx_compute(ebX, gxX):
                for r in range(KS):
                    for c8 in range(8):
                        sl = pl.ds(c8 * L, L)
                        gxX[r, sl] = ebX[r, 0, sl] + off

            def gath_fire(gxX, rowsX):
                for j in range(KS):
                    pltpu.async_copy(tab_h.at[gxX.at[j]],
                                     rowsX.at[pl.ds(j * 128, 128)], gsem)

            def gath_drain(gxX, rowsX):
                for j in range(KS):
                    pltpu.make_async_copy(tab_h.at[gxX.at[j]],
                                          rowsX.at[pl.ds(j * 128, 128)],
                                          gsem).wait()

            def scat_fire(ebX, vbX, rowsX):
                for j in range(KS):
                    pltpu.async_copy(rowsX.at[pl.ds(j * 128, 128)],
                                     acc.at[ebX.at[j, 1]], ssem, add=True)
                if normalize and gl == 0:
                    for j in range(KS):
                        pltpu.async_copy(vbX.at[pl.ds(j * 128, 128)],
                                         rsacc.at[ebX.at[j, 1]], ssem,
                                         add=True)

            def scat_drain(ebX, vbX, rowsX):
                for j in range(KS):
                    pltpu.make_async_copy(rowsX.at[pl.ds(j * 128, 128)],
                                          acc.at[ebX.at[j, 1]], ssem).wait()
                if normalize and gl == 0:
                    for j in range(KS):
                        pltpu.make_async_copy(vbX.at[pl.ds(j * 128, 128)],
                                              rsacc.at[ebX.at[j, 1]],
                                              ssem).wait()

            def scale(vbX, rowsX):
                def sc16(b2, _):
                    rb16 = b2 * L
                    vv = vbX[pl.ds(rb16, L)]
                    for jj in range(L):
                        sp = _splat(vv, cols[jj])
                        rowsX[rb16 + jj, :] = rowsX[rb16 + jj, :] * sp
                    return 0
                lax.fori_loop(0, SUP // L, sc16, 0)

            stage_fire(0, eb0, vb0)
            stage_drain(0, eb0, vb0)
            gidx_compute(eb0, gx0)
            gath_fire(gx0, rows0)

            def pair(t, _):
                i0 = 2 * t
                i1 = i0 + 1
                @pl.when(t > 0)
                def _():
                    scat_drain(eb1, vb1, rows1)
                stage_fire(i1, eb1, vb1)
                gath_drain(gx0, rows0)
                stage_drain(i1, eb1, vb1)
                gidx_compute(eb1, gx1)
                gath_fire(gx1, rows1)
                scale(vb0, rows0)
                scat_fire(eb0, vb0, rows0)
                gath_drain(gx1, rows1)
                scat_drain(eb0, vb0, rows0)

                @pl.when(t < npairs - 1)
                def _():
                    stage_fire(i0 + 2, eb0, vb0)
                    stage_drain(i0 + 2, eb0, vb0)
                    gidx_compute(eb0, gx0)
                    gath_fire(gx0, rows0)
                scale(vb1, rows1)
                scat_fire(eb1, vb1, rows1)
                return 0
            lax.fori_loop(0, npairs, pair, 0)
            scat_drain(eb1, vb1, rows1)
            plsc.subcore_barrier()

            if batch is not None:
                for h in range(2):
                    pltpu.sync_copy(uid_h.at[pl.ds(s_ * 256 + h * 128, 128)],
                                    bidx.at[3])
                    pltpu.sync_copy(acc.at[bidx.at[3]],
                                    rows0.at[pl.ds(h * 128, 128)])
                    pltpu.sync_copy(rsacc.at[bidx.at[3]],
                                    vb0.at[pl.ds(0, 128)])

                    def dvb(b2, _):
                        rb16 = b2 * L
                        rsv = 1.0 / jnp.maximum(vb0[pl.ds(rb16, L)], 1.0)
                        base = h * 128 + rb16
                        for jj in range(L):
                            sp = _splat(rsv, cols[jj])
                            rows0[base + jj, :] = rows0[base + jj, :] * sp
                        return 0
                    lax.fori_loop(0, 8, dvb, 0)
                    pltpu.sync_copy(rows0.at[pl.ds(h * 128, 128)],
                                    ocu.at[g, pl.ds(s_ * 256 + h * 128, 128)])
            else:
                for k in range(wb_iters):
                    idx = s_ + NS * k

                    @pl.when(idx < n_zch)
                    def _():
                        pltpu.sync_copy(acc.at[pl.ds(idx * zch, zch)],
                                        out_h.at[g, pl.ds(idx * zch, zch)])
            plsc.subcore_barrier()

    if batch is None:
        out_type = jax.ShapeDtypeStruct((4, n_rows, L), jnp.float32)
    else:
        out_type = (
            jax.ShapeDtypeStruct((4, B, L), jnp.float32),
            jax.ShapeDtypeStruct((4, B, L), jnp.float32),
            jax.ShapeDtypeStruct((4, B, L), jnp.float32),
            jax.ShapeDtypeStruct((4, B, L), jnp.float32),
            jax.ShapeDtypeStruct((B, SEMD), jnp.float32),
        )
    mesh = plsc.VectorSubcoreMesh(**_MESH)
    kern = pl.kernel(
        body,
        out_type=out_type,
        mesh=mesh,
        scratch_types=scratch,
        compiler_params=pltpu.CompilerParams(use_tc_tiling_on_sc=False),
        name=f"spmm_sc_{n_rows}_{nsuper}",
    )
    if batch is None:
        return kern(pk2, val1, table_flat)
    return kern(pk2, val1, table_flat, *batch)


def _batch_gather(all_flat, sem_tab, c_flat, uid, pid, nid):
    rows_per = B // (NC * NS)

    scratch = [
        pltpu.VMEM((17, 128), jnp.int32),
        pltpu.VMEM((4, rows_per, L), jnp.float32),
        pltpu.VMEM((4, rows_per, L), jnp.float32),
        pltpu.VMEM((4, rows_per, L), jnp.float32),
        pltpu.VMEM((4, rows_per, L), jnp.float32),
        pltpu.VMEM((rows_per, SEMD), jnp.float32),
        pltpu.SemaphoreType.DMA,
    ]

    def body(all_h, sem_h, c_h, uid_h, pid_h, nid_h,
             oeu, opos, oneg, ocu, oz, idxb, eub, posb, negb, cub, zb, sem):
        c = lax.axis_index("c")
        s = lax.axis_index("s")
        wid = s * NC + c
        base = wid * rows_per
        pltpu.sync_copy(uid_h.at[pl.ds(base, 128)], idxb.at[16])
        pltpu.sync_copy(pid_h.at[pl.ds(base, 128)], idxb.at[4])
        pltpu.sync_copy(nid_h.at[pl.ds(base, 128)], idxb.at[8])
        for g in range(4):
            for c8 in range(8):
                sl = pl.ds(c8 * L, L)
                u = idxb[16, sl]
                idxb[g, sl] = u + g * N
                idxb[12 + g, sl] = u + g * NU
                if g > 0:
                    idxb[4 + g, sl] = idxb[4, sl] + g * N
                    idxb[8 + g, sl] = idxb[8, sl] + g * N
        ds = []
        for g in range(4):
            ds.append(pltpu.async_copy(all_h.at[idxb.at[g]], eub.at[g], sem))
            ds.append(pltpu.async_copy(all_h.at[idxb.at[4 + g]], posb.at[g],
                                       sem))
            ds.append(pltpu.async_copy(all_h.at[idxb.at[8 + g]], negb.at[g],
                                       sem))
            ds.append(pltpu.async_copy(c_h.at[idxb.at[12 + g]], cub.at[g],
                                       sem))
        ds.append(pltpu.async_copy(sem_h.at[idxb.at[16]], zb, sem))
        for d in ds:
            d.wait()
        for g in range(4):
            pltpu.sync_copy(eub.at[g], oeu.at[g, pl.ds(base, rows_per)])
            pltpu.sync_copy(posb.at[g], opos.at[g, pl.ds(base, rows_per)])
            pltpu.sync_copy(negb.at[g], oneg.at[g, pl.ds(base, rows_per)])
            pltpu.sync_copy(cub.at[g], ocu.at[g, pl.ds(base, rows_per)])
        pltpu.sync_copy(zb, oz.at[pl.ds(base, rows_per)])

    mesh = plsc.VectorSubcoreMesh(**_MESH)
    kern = pl.kernel(
        body,
        out_type=(
            jax.ShapeDtypeStruct((4, B, L), jnp.float32),
            jax.ShapeDtypeStruct((4, B, L), jnp.float32),
            jax.ShapeDtypeStruct((4, B, L), jnp.float32),
            jax.ShapeDtypeStruct((4, B, L), jnp.float32),
            jax.ShapeDtypeStruct((B, SEMD), jnp.float32),
        ),
        mesh=mesh,
        scratch_types=scratch,
        compiler_params=pltpu.CompilerParams(use_tc_tiling_on_sc=False),
        name="batch_gather_sc",
    )
    return kern(all_flat, sem_tab, c_flat, uid, pid, nid)


def _mean3(a, b, c):
    R = a.shape[0]
    blk = 2000

    def body(ar, br, cr, orr):
        orr[...] = (ar[...] + br[...] + cr[...]) * (1.0 / 3.0)

    return pl.pallas_call(
        body,
        grid=(R // blk,),
        in_specs=[pl.BlockSpec((blk, 128), lambda i: (i, 0))] * 3,
        out_specs=pl.BlockSpec((blk, 128), lambda i: (i, 0)),
        out_shape=jax.ShapeDtypeStruct((R, 128), jnp.float32),
    )(a, b, c)


def _tail(eu, pos, neg, cu, z, W_proj, b_proj, W_gate, b_gate):
    BB = 512

    def body(eu_r, pos_r, neg_r, cu_r, z_r, wp_r, bp_r, wg_r, bg_r, o_r):
        e = jnp.concatenate([eu_r[j] for j in range(4)], axis=-1)
        p = jnp.concatenate([pos_r[j] for j in range(4)], axis=-1)
        n = jnp.concatenate([neg_r[j] for j in range(4)], axis=-1)
        cc = jnp.concatenate([cu_r[j] for j in range(4)], axis=-1)
        wg = wg_r[...]
        delta = (jnp.dot(z_r[...], wp_r[...],
                         preferred_element_type=jnp.float32) + bp_r[...])
        h = (jnp.dot(e, wg[0:64], preferred_element_type=jnp.float32)
             + jnp.dot(cc, wg[64:128], preferred_element_type=jnp.float32)
             + jnp.dot(delta, wg[128:192], preferred_element_type=jnp.float32)
             + bg_r[...])
        gate = jax.nn.sigmoid(h)
        ue = e + gate * delta
        ps = jnp.sum(ue * p, axis=1)
        ns = jnp.sum(ue * n, axis=1)
        o_r[0:1, :] = ps.reshape(1, BB)
        o_r[1:2, :] = ns.reshape(1, BB)

    pm = pl.BlockSpec((4, BB, L), lambda i: (0, i, 0))
    return pl.pallas_call(
        body,
        grid=(B // BB,),
        in_specs=[
            pm, pm, pm, pm,
            pl.BlockSpec((BB, SEMD), lambda i: (i, 0)),
            pl.BlockSpec((SEMD, D), lambda i: (0, 0)),
            pl.BlockSpec((1, D), lambda i: (0, 0)),
            pl.BlockSpec((3 * D, D), lambda i: (0, 0)),
            pl.BlockSpec((1, D), lambda i: (0, 0)),
        ],
        out_specs=pl.BlockSpec((2, BB), lambda i: (0, i)),
        out_shape=jax.ShapeDtypeStruct((2, B), jnp.float32),
    )(eu, pos, neg, cu, z, W_proj, b_proj, W_gate, b_gate)


def _pad_edges(dst, src, val, e_pad, n_dst, n_src):
    e = dst.shape[0]
    pad = e_pad - e
    ar = jnp.arange(pad, dtype=jnp.int32)
    dst = jnp.concatenate([dst.astype(jnp.int32), ar % n_dst])
    src = jnp.concatenate([src.astype(jnp.int32), ar % n_src])
    val = jnp.concatenate([val, jnp.zeros((pad,), jnp.float32)])
    return jnp.stack([src.reshape(e_pad // 128, 128),
                      dst.reshape(e_pad // 128, 128)], axis=1), val


def kernel(adj_indices, adj_values, ui_indices, ui_values, user_ids,
           pos_item_ids, neg_item_ids, emb_user, emb_item, sem_table,
           W_proj, b_proj, W_gate, b_gate):
    EA_P = 16 * 148 * 512
    EU_P = 16 * 76 * 512
    pkA, valA = _pad_edges(adj_indices[0], adj_indices[1], adj_values,
                           EA_P, N, N)
    e0 = jnp.concatenate([emb_user, emb_item], axis=0)
    e0_pl = e0.reshape(N, 4, L).transpose(1, 0, 2)
    e1_pl = _spmm_planes(pkA, valA, e0_pl.reshape(4 * N, L), N, 148, 400, N,
                         False)
    e2_pl = _spmm_planes(pkA, valA, e1_pl.reshape(4 * N, L), N, 148, 400, N,
                         False)
    all_pl = _mean3(e0_pl.reshape(-1, 128), e1_pl.reshape(-1, 128),
                    e2_pl.reshape(-1, 128)).reshape(4 * N, L)

    pkU, valU = _pad_edges(ui_indices[0],
                           ui_indices[1].astype(jnp.int32) + NU,
                           ui_values, EU_P, NU, N)
    uid = user_ids.astype(jnp.int32)
    pid = pos_item_ids.astype(jnp.int32) + NU
    nid = neg_item_ids.astype(jnp.int32) + NU
    eu, pos, neg, cu, z = _spmm_planes(pkU, valU, all_pl, NU, 76, 400, N,
                                       True,
                                       batch=(sem_table, uid, pid, nid))

    return _tail(eu, pos, neg, cu, z, W_proj, b_proj.reshape(1, D),
                 W_gate, b_gate.reshape(1, D))


# speedup vs baseline: 8.9939x; 1.0093x over previous
import jax
import jax.numpy as jnp
from jax import lax
from jax.experimental import pallas as pl
from jax.experimental.pallas import tpu as pltpu
from jax.experimental.pallas import tpu_sc as plsc

NU = 50000
NI = 50000
N = NU + NI
D = 64
SEMD = 256
B = 4096
NC = 2
NS = 16
L = 16

CHUNK = 1024
KSUB = CHUNK // 128

_MESH = dict(core_axis_name="c", subcore_axis_name="s", num_cores=NC,
             num_subcores=NS)


def _splat(vec, idx):
    dn = lax.GatherDimensionNumbers(offset_dims=(), collapsed_slice_dims=(0,),
                                    start_index_map=(0,))
    return lax.gather(vec, idx[:, None], dn, slice_sizes=(1,),
                      mode=lax.GatherScatterMode.PROMISE_IN_BOUNDS)


def _spmm_planes(pk2, val1, table_flat, n_rows, nsuper, zch, plane_rows,
                 normalize, batch=None):
    n_zch = n_rows // zch
    wb_iters = (n_zch + NS - 1) // NS
    SUP = 512
    KS = SUP // 128
    ept = nsuper * SUP
    npairs = nsuper // 2

    scratch = [
        pltpu.VMEM_SHARED((n_rows, L), jnp.float32),
        pltpu.VMEM((SUP, L), jnp.float32),
        pltpu.VMEM((SUP, L), jnp.float32),
        pltpu.VMEM((4, 2, 128), jnp.int32),
        pltpu.VMEM((4, 2, 128), jnp.int32),
        pltpu.VMEM((SUP,), jnp.float32),
        pltpu.VMEM((SUP,), jnp.float32),
        pltpu.VMEM((4, 128), jnp.int32),
        pltpu.VMEM((4, 128), jnp.int32),
        pltpu.SemaphoreType.DMA,
        pltpu.SemaphoreType.DMA,
        pltpu.SemaphoreType.DMA,
    ]
    if normalize:
        scratch += [
            pltpu.VMEM_SHARED((n_rows,), jnp.float32),
            pltpu.VMEM((zch,), jnp.float32),
        ]
    if batch is not None:
        scratch += [
            pltpu.VMEM((4, 128), jnp.int32),
            pltpu.VMEM((128, SEMD), jnp.float32),
        ]

    def body(*refs):
        it = iter(refs)
        pk_h, val_h, tab_h = next(it), next(it), next(it)
        if batch is not None:
            sem_h, uid_h, pid_h, nid_h = next(it), next(it), next(it), next(it)
            oeu, opos, oneg, ocu, oz = (next(it), next(it), next(it),
                                        next(it), next(it))
        else:
            out_h = next(it)
        acc, rows0, rows1 = next(it), next(it), next(it)
        eb0, eb1, vb0, vb1, gx0, gx1 = (next(it), next(it), next(it),
                                        next(it), next(it), next(it))
        stsem, gsem, ssem = next(it), next(it), next(it)
        if normalize:
            rsacc, rsbuf = next(it), next(it)
        if batch is not None:
            bidx, zb = next(it), next(it)
        c = lax.axis_index("c")
        s_ = lax.axis_index("s")
        iota = lax.iota(jnp.int32, L)
        cols = [iota * 0 + j for j in range(L)]

        if batch is not None:
            wid = s_ * NC + c
            wbase = wid * 128
            pltpu.sync_copy(uid_h.at[pl.ds(wbase, 128)], bidx.at[0])
            pltpu.sync_copy(pid_h.at[pl.ds(wbase, 128)], bidx.at[1])
            pltpu.sync_copy(nid_h.at[pl.ds(wbase, 128)], bidx.at[2])
            dz = pltpu.async_copy(sem_h.at[bidx.at[0]], zb, gsem)
            for tix, obuf in ((0, oeu), (1, opos), (2, oneg)):
                for g4 in range(4):
                    for c8 in range(8):
                        sl = pl.ds(c8 * L, L)
                        gx0[g4, sl] = bidx[tix, sl] + g4 * plane_rows
                for g4 in range(4):
                    pltpu.sync_copy(tab_h.at[gx0.at[g4]],
                                    rows0.at[pl.ds(g4 * 128, 128)])
                for g4 in range(4):
                    pltpu.sync_copy(rows0.at[pl.ds(g4 * 128, 128)],
                                    obuf.at[g4, pl.ds(wbase, 128)])
            dz.wait()
            pltpu.sync_copy(zb, oz.at[pl.ds(wbase, 128)])

        for gl in range(2):
            g = c * 2 + gl
            off = g * plane_rows

            def zrow(i, _):
                rows0[i, :] = jnp.zeros((L,), jnp.float32)
                return 0
            lax.fori_loop(0, zch, zrow, 0)
            if normalize and gl == 0:
                def zrs(i, _):
                    rsbuf[pl.ds(i * L, L)] = jnp.zeros((L,), jnp.float32)
                    return 0
                lax.fori_loop(0, zch // L, zrs, 0)
            for k in range(wb_iters):
                idx = s_ + NS * k

                @pl.when(idx < n_zch)
                def _():
                    pltpu.async_copy(rows0.at[pl.ds(0, zch)],
                                     acc.at[pl.ds(idx * zch, zch)], stsem)
                    if normalize and gl == 0:
                        pltpu.async_copy(rsbuf,
                                         rsacc.at[pl.ds(idx * zch, zch)],
                                         stsem)
            for k in range(wb_iters):
                idx = s_ + NS * k

                @pl.when(idx < n_zch)
                def _():
                    pltpu.make_async_copy(rows0.at[pl.ds(0, zch)],
                                          acc.at[pl.ds(idx * zch, zch)],
                                          stsem).wait()
                    if normalize and gl == 0:
                        pltpu.make_async_copy(rsbuf,
                                              rsacc.at[pl.ds(idx * zch, zch)],
                                              stsem).wait()
            plsc.subcore_barrier()

            def stage_fire(i, ebX, vbX):
                rb = s_ * (ept // 128) + i * KS
                pltpu.async_copy(pk_h.at[pl.ds(rb, KS)], ebX, stsem)
                pltpu.async_copy(val_h.at[pl.ds(s_ * ept + i * SUP, SUP)],
                                 vbX, stsem)

            def stage_drain(i, ebX, vbX):
                rb = s_ * (ept // 128) + i * KS
                pltpu.make_async_copy(pk_h.at[pl.ds(rb, KS)], ebX,
                                      stsem).wait()
                pltpu.make_async_copy(
                    val_h.at[pl.ds(s_ * ept + i * SUP, SUP)], vbX,
                    stsem).wait()

            def gidx_compute(ebX, gxX):
                for r in range(KS):
                    for c8 in range(8):
                        sl = pl.ds(c8 * L, L)
                        gxX[r, sl] = ebX[r, 0, sl] + off

            def gath_fire(gxX, rowsX):
                for j in range(KS):
                    pltpu.async_copy(tab_h.at[gxX.at[j]],
                                     rowsX.at[pl.ds(j * 128, 128)], gsem)

            def gath_drain(gxX, rowsX):
                for j in range(KS):
                    pltpu.make_async_copy(tab_h.at[gxX.at[j]],
                                          rowsX.at[pl.ds(j * 128, 128)],
                                          gsem).wait()

            def scat_fire(ebX, vbX, rowsX):
                for j in range(KS):
                    pltpu.async_copy(rowsX.at[pl.ds(j * 128, 128)],
                                     acc.at[ebX.at[j, 1]], ssem, add=True)
                if normalize and gl == 0:
                    for j in range(KS):
                        pltpu.async_copy(vbX.at[pl.ds(j * 128, 128)],
                                         rsacc.at[ebX.at[j, 1]], ssem,
                                         add=True)

            def scat_drain(ebX, vbX, rowsX):
                for j in range(KS):
                    pltpu.make_async_copy(rowsX.at[pl.ds(j * 128, 128)],
                                          acc.at[ebX.at[j, 1]], ssem).wait()
                if normalize and gl == 0:
                    for j in range(KS):
                        pltpu.make_async_copy(vbX.at[pl.ds(j * 128, 128)],
                                              rsacc.at[ebX.at[j, 1]],
                                              ssem).wait()

            def scale(vbX, rowsX):
                def sc16(b2, _):
                    rb16 = b2 * L
                    vv = vbX[pl.ds(rb16, L)]
                    for jj in range(L):
                        sp = _splat(vv, cols[jj])
                        rowsX[rb16 + jj, :] = rowsX[rb16 + jj, :] * sp
                    return 0
                lax.fori_loop(0, SUP // L, sc16, 0)

            stage_fire(0, eb0, vb0)
            stage_drain(0, eb0, vb0)
            gidx_compute(eb0, gx0)
            gath_fire(gx0, rows0)

            def pair(t, _):
                i0 = 2 * t
                i1 = i0 + 1
                @pl.when(t > 0)
                def _():
                    scat_drain(eb1, vb1, rows1)
                stage_fire(i1, eb1, vb1)
                gath_drain(gx0, rows0)
                stage_drain(i1, eb1, vb1)
                gidx_compute(eb1, gx1)
                gath_fire(gx1, rows1)
                scale(vb0, rows0)
                scat_fire(eb0, vb0, rows0)
                gath_drain(gx1, rows1)
                scat_drain(eb0, vb0, rows0)

                @pl.when(t < npairs - 1)
                def _():
                    stage_fire(i0 + 2, eb0, vb0)
                    stage_drain(i0 + 2, eb0, vb0)
                    gidx_compute(eb0, gx0)
                    gath_fire(gx0, rows0)
                scale(vb1, rows1)
                scat_fire(eb1, vb1, rows1)
                return 0
            lax.fori_loop(0, npairs, pair, 0)
            scat_drain(eb1, vb1, rows1)
            plsc.subcore_barrier()

            if batch is not None:
                for h in range(2):
                    pltpu.sync_copy(uid_h.at[pl.ds(s_ * 256 + h * 128, 128)],
                                    bidx.at[3])
                    pltpu.sync_copy(acc.at[bidx.at[3]],
                                    rows0.at[pl.ds(h * 128, 128)])
                    pltpu.sync_copy(rsacc.at[bidx.at[3]],
                                    vb0.at[pl.ds(0, 128)])

                    def dvb(b2, _):
                        rb16 = b2 * L
                        rsv = 1.0 / jnp.maximum(vb0[pl.ds(rb16, L)], 1.0)
                        base = h * 128 + rb16
                        for jj in range(L):
                            sp = _splat(rsv, cols[jj])
                            rows0[base + jj, :] = rows0[base + jj, :] * sp
                        return 0
                    lax.fori_loop(0, 8, dvb, 0)
                    pltpu.sync_copy(rows0.at[pl.ds(h * 128, 128)],
                                    ocu.at[g, pl.ds(s_ * 256 + h * 128, 128)])
            else:
                for k in range(wb_iters):
                    idx = s_ + NS * k

                    @pl.when(idx < n_zch)
                    def _():
                        pltpu.async_copy(acc.at[pl.ds(idx * zch, zch)],
                                         out_h.at[g, pl.ds(idx * zch, zch)],
                                         stsem)
                for k in range(wb_iters):
                    idx = s_ + NS * k

                    @pl.when(idx < n_zch)
                    def _():
                        pltpu.make_async_copy(
                            acc.at[pl.ds(idx * zch, zch)],
                            out_h.at[g, pl.ds(idx * zch, zch)], stsem).wait()
            plsc.subcore_barrier()

    if batch is None:
        out_type = jax.ShapeDtypeStruct((4, n_rows, L), jnp.float32)
    else:
        out_type = (
            jax.ShapeDtypeStruct((4, B, L), jnp.float32),
            jax.ShapeDtypeStruct((4, B, L), jnp.float32),
            jax.ShapeDtypeStruct((4, B, L), jnp.float32),
            jax.ShapeDtypeStruct((4, B, L), jnp.float32),
            jax.ShapeDtypeStruct((B, SEMD), jnp.float32),
        )
    mesh = plsc.VectorSubcoreMesh(**_MESH)
    kern = pl.kernel(
        body,
        out_type=out_type,
        mesh=mesh,
        scratch_types=scratch,
        compiler_params=pltpu.CompilerParams(use_tc_tiling_on_sc=False),
        name=f"spmm_sc_{n_rows}_{nsuper}",
    )
    if batch is None:
        return kern(pk2, val1, table_flat)
    return kern(pk2, val1, table_flat, *batch)


def _mean3(a, b, c):
    R = a.shape[0]
    blk = 2000

    def body(ar, br, cr, orr):
        orr[...] = (ar[...] + br[...] + cr[...]) * (1.0 / 3.0)

    return pl.pallas_call(
        body,
        grid=(R // blk,),
        in_specs=[pl.BlockSpec((blk, 128), lambda i: (i, 0))] * 3,
        out_specs=pl.BlockSpec((blk, 128), lambda i: (i, 0)),
        out_shape=jax.ShapeDtypeStruct((R, 128), jnp.float32),
    )(a, b, c)


def _tail(eu, pos, neg, cu, z, W_proj, b_proj, W_gate, b_gate):
    BB = 512

    def body(eu_r, pos_r, neg_r, cu_r, z_r, wp_r, bp_r, wg_r, bg_r, o_r):
        e = jnp.concatenate([eu_r[j] for j in range(4)], axis=-1)
        p = jnp.concatenate([pos_r[j] for j in range(4)], axis=-1)
        n = jnp.concatenate([neg_r[j] for j in range(4)], axis=-1)
        cc = jnp.concatenate([cu_r[j] for j in range(4)], axis=-1)
        wg = wg_r[...]
        delta = (jnp.dot(z_r[...], wp_r[...],
                         preferred_element_type=jnp.float32) + bp_r[...])
        h = (jnp.dot(e, wg[0:64], preferred_element_type=jnp.float32)
             + jnp.dot(cc, wg[64:128], preferred_element_type=jnp.float32)
             + jnp.dot(delta, wg[128:192], preferred_element_type=jnp.float32)
             + bg_r[...])
        gate = jax.nn.sigmoid(h)
        ue = e + gate * delta
        ps = jnp.sum(ue * p, axis=1)
        ns = jnp.sum(ue * n, axis=1)
        o_r[0:1, :] = ps.reshape(1, BB)
        o_r[1:2, :] = ns.reshape(1, BB)

    pm = pl.BlockSpec((4, BB, L), lambda i: (0, i, 0))
    return pl.pallas_call(
        body,
        grid=(B // BB,),
        in_specs=[
            pm, pm, pm, pm,
            pl.BlockSpec((BB, SEMD), lambda i: (i, 0)),
            pl.BlockSpec((SEMD, D), lambda i: (0, 0)),
            pl.BlockSpec((1, D), lambda i: (0, 0)),
            pl.BlockSpec((3 * D, D), lambda i: (0, 0)),
            pl.BlockSpec((1, D), lambda i: (0, 0)),
        ],
        out_specs=pl.BlockSpec((2, BB), lambda i: (0, i)),
        out_shape=jax.ShapeDtypeStruct((2, B), jnp.float32),
    )(eu, pos, neg, cu, z, W_proj, b_proj, W_gate, b_gate)


def _pad_edges(dst, src, val, e_pad, n_dst, n_src):
    e = dst.shape[0]
    pad = e_pad - e
    ar = jnp.arange(pad, dtype=jnp.int32)
    dst = jnp.concatenate([dst.astype(jnp.int32), ar % n_dst])
    src = jnp.concatenate([src.astype(jnp.int32), ar % n_src])
    val = jnp.concatenate([val, jnp.zeros((pad,), jnp.float32)])
    return jnp.stack([src.reshape(e_pad // 128, 128),
                      dst.reshape(e_pad // 128, 128)], axis=1), val


def kernel(adj_indices, adj_values, ui_indices, ui_values, user_ids,
           pos_item_ids, neg_item_ids, emb_user, emb_item, sem_table,
           W_proj, b_proj, W_gate, b_gate):
    EA_P = 16 * 148 * 512
    EU_P = 16 * 76 * 512
    pkA, valA = _pad_edges(adj_indices[0], adj_indices[1], adj_values,
                           EA_P, N, N)
    e0 = jnp.concatenate([emb_user, emb_item], axis=0)
    e0_pl = e0.reshape(N, 4, L).transpose(1, 0, 2)
    e1_pl = _spmm_planes(pkA, valA, e0_pl.reshape(4 * N, L), N, 148, 400, N,
                         False)
    e2_pl = _spmm_planes(pkA, valA, e1_pl.reshape(4 * N, L), N, 148, 400, N,
                         False)
    all_pl = _mean3(e0_pl.reshape(-1, 128), e1_pl.reshape(-1, 128),
                    e2_pl.reshape(-1, 128)).reshape(4 * N, L)

    pkU, valU = _pad_edges(ui_indices[0],
                           ui_indices[1].astype(jnp.int32) + NU,
                           ui_values, EU_P, NU, N)
    uid = user_ids.astype(jnp.int32)
    pid = pos_item_ids.astype(jnp.int32) + NU
    nid = neg_item_ids.astype(jnp.int32) + NU
    eu, pos, neg, cu, z = _spmm_planes(pkU, valU, all_pl, NU, 76, 400, N,
                                       True,
                                       batch=(sem_table, uid, pid, nid))

    return _tail(eu, pos, neg, cu, z, W_proj, b_proj.reshape(1, D),
                 W_gate, b_gate.reshape(1, D))
